# baseline (speedup 1.0000x reference)
import jax
import jax.numpy as jnp
from jax.experimental import pallas as pl
from jax.experimental.pallas import tpu as pltpu

_BATCH = 16
_N_NODES = 512
_HIDDEN = 768
_BB = 16


def _tc_body(node_ref, tok_ref, out_ref):
    for i in range(_BB):
        out_ref[i, 0:1, :] = tok_ref[...]
        out_ref[i, 1:_N_NODES + 1, :] = node_ref[i]


@jax.jit
def _tc_call(node_feature, graph_token):
    return pl.pallas_call(
        _tc_body,
        grid=(_BATCH // _BB,),
        in_specs=[
            pl.BlockSpec((_BB, _N_NODES, _HIDDEN), lambda b: (b, 0, 0)),
            pl.BlockSpec((1, _HIDDEN), lambda b: (0, 0)),
        ],
        out_specs=pl.BlockSpec((_BB, _N_NODES + 1, _HIDDEN),
                               lambda b: (b, 0, 0)),
        out_shape=jax.ShapeDtypeStruct((_BATCH, _N_NODES + 1, _HIDDEN),
                                       jnp.float32),
        compiler_params=pltpu.CompilerParams(
            dimension_semantics=("parallel",),
        ),
    )(node_feature, graph_token)


def kernel(node_feature, graph_token):
    return _tc_call(node_feature, graph_token)
